# Initial kernel scaffold; baseline (speedup 1.0000x reference)
import jax
import jax.numpy as jnp
from jax.experimental import pallas as pl


def kernel(x, tok_embed, pos_embed, gamma, beta):
    raise NotImplementedError("write your pallas kernel here")


# speedup vs baseline: 2.2761x; 2.2761x over previous
import functools

import jax
import jax.numpy as jnp
from jax import lax
from jax.experimental import pallas as pl
from jax.experimental.pallas import tpu as pltpu
from jax.experimental.pallas import tpu_sc as plsc

D = 64
SEQ = 200
BATCH = 4096
NTOK = BATCH * SEQ

NC = 2
NS = 16
NW = NC * NS
TOK_PER_W = NTOK // NW
CHUNK = SEQ
N_CHUNKS = TOK_PER_W // CHUNK


def _rsqrt_vec(v):
    i = plsc.bitcast(v, jnp.int32)
    y = plsc.bitcast(jnp.full((16,), 0x5F3759DF, jnp.int32) - (i >> 1),
                     jnp.float32)
    y = y * (1.5 - 0.5 * v * y * y)
    y = y * (1.5 - 0.5 * v * y * y)
    y = y * (1.5 - 0.5 * v * y * y)
    return y


def _make_sc_kernel():
    mesh = plsc.VectorSubcoreMesh(core_axis_name="c", subcore_axis_name="s")

    @functools.partial(
        pl.kernel,
        mesh=mesh,
        compiler_params=pltpu.CompilerParams(
            needs_layout_passes=False, use_tc_tiling_on_sc=False),
        out_type=jax.ShapeDtypeStruct((NTOK, D), jnp.float32),
        scratch_types=[
            pltpu.VMEM((CHUNK,), jnp.int32),
            pltpu.VMEM((CHUNK, D), jnp.float32),
            pltpu.VMEM((CHUNK, D), jnp.float32),
            pltpu.VMEM((SEQ, D), jnp.float32),
            pltpu.VMEM((D,), jnp.float32),
            pltpu.VMEM((D,), jnp.float32),
            pltpu.SemaphoreType.DMA,
        ],
    )
    def emb_kernel(xf_hbm, tok_hbm, pos_hbm, g_hbm, b_hbm, out_hbm,
                   idx_v, rows_v, out_v, pos_v, g_v, b_v, sem):
        wid = lax.axis_index("s") * NC + lax.axis_index("c")
        pltpu.sync_copy(pos_hbm, pos_v)
        pltpu.sync_copy(g_hbm, g_v)
        pltpu.sync_copy(b_hbm, b_v)

        g = [g_v[pl.ds(16 * k, 16)] for k in range(4)]
        b = [b_v[pl.ds(16 * k, 16)] for k in range(4)]
        base0 = wid * TOK_PER_W

        def chunk_body(cid, carry):
            base = pl.multiple_of(base0 + cid * CHUNK, 8)
            pltpu.sync_copy(xf_hbm.at[pl.ds(base, CHUNK)], idx_v)
            pltpu.async_copy(tok_hbm.at[idx_v], rows_v, sem).wait()

            def tok_body(t, c2):
                h = [rows_v[t, pl.ds(16 * k, 16)] + pos_v[t, pl.ds(16 * k, 16)]
                     for k in range(4)]
                tot = jnp.sum((h[0] + h[1]) + (h[2] + h[3]))
                mean = tot * (1.0 / D)
                ssq = jnp.sum((h[0] * h[0] + h[1] * h[1])
                              + (h[2] * h[2] + h[3] * h[3]))
                var = ssq * (1.0 / D) - mean * mean
                rstd = _rsqrt_vec(jnp.full((16,), var + 1e-5, jnp.float32))
                for k in range(4):
                    out_v[t, pl.ds(16 * k, 16)] = (h[k] - mean) * rstd * g[k] + b[k]
                return c2

            lax.fori_loop(0, CHUNK, tok_body, 0)
            pltpu.sync_copy(out_v, out_hbm.at[pl.ds(base, CHUNK)])
            return carry

        lax.fori_loop(0, N_CHUNKS, chunk_body, 0)

    return emb_kernel


_emb_kernel = _make_sc_kernel()


@jax.jit
def kernel(x, tok_embed, pos_embed, gamma, beta):
    xf = x.reshape(-1).astype(jnp.int32)
    out = _emb_kernel(xf, tok_embed, pos_embed, gamma, beta)
    return out.reshape(BATCH, SEQ, D)


# speedup vs baseline: 2.7641x; 1.2144x over previous
import functools

import jax
import jax.numpy as jnp
from jax import lax
from jax.experimental import pallas as pl
from jax.experimental.pallas import tpu as pltpu
from jax.experimental.pallas import tpu_sc as plsc

D = 64
SEQ = 200
BATCH = 4096
NTOK = BATCH * SEQ

NC = 2
NS = 16
NW = NC * NS
TOK_PER_W = NTOK // NW
CHUNK = SEQ
N_CHUNKS = TOK_PER_W // CHUNK
NBUF = 2


def _rsqrt_vec(v):
    i = plsc.bitcast(v, jnp.int32)
    y = plsc.bitcast(jnp.full((16,), 0x5F3759DF, jnp.int32) - (i >> 1),
                     jnp.float32)
    y = y * (1.5 - 0.5 * v * y * y)
    y = y * (1.5 - 0.5 * v * y * y)
    y = y * (1.5 - 0.5 * v * y * y)
    return y


def _make_sc_kernel():
    mesh = plsc.VectorSubcoreMesh(core_axis_name="c", subcore_axis_name="s")

    @functools.partial(
        pl.kernel,
        mesh=mesh,
        compiler_params=pltpu.CompilerParams(
            needs_layout_passes=False, use_tc_tiling_on_sc=False),
        out_type=jax.ShapeDtypeStruct((NTOK, D), jnp.float32),
        scratch_types=[
            pltpu.VMEM((TOK_PER_W,), jnp.int32),
            pltpu.VMEM((NBUF, CHUNK, D), jnp.float32),
            pltpu.VMEM((NBUF, CHUNK, D), jnp.float32),
            pltpu.VMEM((SEQ, D), jnp.float32),
            pltpu.VMEM((D,), jnp.float32),
            pltpu.VMEM((D,), jnp.float32),
            pltpu.SemaphoreType.DMA,
            pltpu.SemaphoreType.DMA,
            pltpu.SemaphoreType.DMA,
            pltpu.SemaphoreType.DMA,
        ],
    )
    def emb_kernel(xf_hbm, tok_hbm, pos_hbm, g_hbm, b_hbm, out_hbm,
                   idx_all, rows_v, out_v, pos_v, g_v, b_v,
                   gsem0, gsem1, osem0, osem1):
        gsem = [gsem0, gsem1]
        osem = [osem0, osem1]
        wid = lax.axis_index("s") * NC + lax.axis_index("c")
        pltpu.sync_copy(pos_hbm, pos_v)
        pltpu.sync_copy(g_hbm, g_v)
        pltpu.sync_copy(b_hbm, b_v)

        g = [g_v[pl.ds(16 * k, 16)] for k in range(4)]
        b = [b_v[pl.ds(16 * k, 16)] for k in range(4)]
        base0 = pl.multiple_of(wid * TOK_PER_W, 8)
        pltpu.sync_copy(xf_hbm.at[pl.ds(base0, TOK_PER_W)], idx_all)

        for bb in range(NBUF):
            pltpu.async_copy(
                tok_hbm.at[idx_all.at[pl.ds(bb * CHUNK, CHUNK)]],
                rows_v.at[bb], gsem[bb])

        def pair_body(p, carry):
            for bb in range(NBUF):
                c = NBUF * p + bb
                pltpu.make_async_copy(
                    tok_hbm.at[idx_all.at[pl.ds(0, CHUNK)]],
                    rows_v.at[bb], gsem[bb]).wait()
                @pl.when(p > 0)
                def _():
                    pltpu.make_async_copy(
                        out_v.at[bb], out_hbm.at[pl.ds(0, CHUNK)],
                        osem[bb]).wait()

                @plsc.parallel_loop(0, CHUNK, 1, unroll=4)
                def tok_body(t):
                    h = [rows_v[bb, t, pl.ds(16 * k, 16)]
                         + pos_v[t, pl.ds(16 * k, 16)] for k in range(4)]
                    tot = jnp.sum((h[0] + h[1]) + (h[2] + h[3]))
                    mean = tot * (1.0 / D)
                    ssq = jnp.sum((h[0] * h[0] + h[1] * h[1])
                                  + (h[2] * h[2] + h[3] * h[3]))
                    var = ssq * (1.0 / D) - mean * mean
                    rstd = _rsqrt_vec(
                        jnp.full((16,), var + 1e-5, jnp.float32))
                    for k in range(4):
                        out_v[bb, t, pl.ds(16 * k, 16)] = (
                            (h[k] - mean) * (rstd * g[k]) + b[k])

                base = pl.multiple_of(base0 + c * CHUNK, 8)
                pltpu.async_copy(
                    out_v.at[bb], out_hbm.at[pl.ds(base, CHUNK)], osem[bb])

                @pl.when(p < (N_CHUNKS // NBUF) - 1)
                def _():
                    off = pl.multiple_of((c + NBUF) * CHUNK, 8)
                    pltpu.async_copy(
                        tok_hbm.at[idx_all.at[pl.ds(off, CHUNK)]],
                        rows_v.at[bb], gsem[bb])
            return carry

        lax.fori_loop(0, N_CHUNKS // NBUF, pair_body, 0)
        for bb in range(NBUF):
            pltpu.make_async_copy(
                out_v.at[bb], out_hbm.at[pl.ds(0, CHUNK)], osem[bb]).wait()

    return emb_kernel


_emb_kernel = _make_sc_kernel()


@jax.jit
def kernel(x, tok_embed, pos_embed, gamma, beta):
    xf = x.reshape(-1).astype(jnp.int32)
    out = _emb_kernel(xf, tok_embed, pos_embed, gamma, beta)
    return out.reshape(BATCH, SEQ, D)
